# Initial kernel scaffold; baseline (speedup 1.0000x reference)
import jax
import jax.numpy as jnp
from jax.experimental import pallas as pl


def kernel(world_pos, mesh_pos, target_world_pos, node_type, edge_index, params):
    raise NotImplementedError("write your pallas kernel here")


# speedup vs baseline: 2.1449x; 2.1449x over previous
import functools

import jax
import jax.numpy as jnp
from jax import lax
from jax.experimental import pallas as pl
from jax.experimental.pallas import tpu as pltpu
from jax.experimental.pallas import tpu_sc as plsc

N = 10000
E = 320000
LAT = 128
STEPS = 15
C = 128
NW = 32
EP = 327680
CH = EP // (NW * C)
NP = 10112
RSUB = NP // 16
f32 = jnp.float32


@functools.lru_cache(maxsize=None)
def _make_gather2(d):
    mesh = plsc.VectorSubcoreMesh(core_axis_name="c", subcore_axis_name="s")

    @functools.partial(
        pl.kernel,
        mesh=mesh,
        out_type=(jax.ShapeDtypeStruct((EP, d), f32),
                  jax.ShapeDtypeStruct((EP, d), f32)),
        scratch_types=[
            pltpu.VMEM((C,), jnp.int32),
            pltpu.VMEM((C,), jnp.int32),
            pltpu.VMEM((C, d), f32),
            pltpu.VMEM((C, d), f32),
            pltpu.SemaphoreType.DMA,
            pltpu.SemaphoreType.DMA,
        ],
    )
    def gk(ta, ia, tb, ib, oa, ob, iva, ivb, ra, rb, sa, sb):
        w = lax.axis_index("s") * 2 + lax.axis_index("c")

        def body(j, carry):
            row = w * CH + j
            pltpu.sync_copy(ia.at[row], iva)
            pltpu.sync_copy(ib.at[row], ivb)
            ca = pltpu.async_copy(ta.at[iva], ra, sa)
            cb = pltpu.async_copy(tb.at[ivb], rb, sb)
            ca.wait()
            cb.wait()
            base = row * C
            pltpu.sync_copy(ra, oa.at[pl.ds(base, C)])
            pltpu.sync_copy(rb, ob.at[pl.ds(base, C)])
            return carry

        lax.fori_loop(0, CH, body, 0)

    return gk


@functools.lru_cache(maxsize=None)
def _make_scatter():
    mesh = plsc.VectorSubcoreMesh(core_axis_name="c", subcore_axis_name="s")

    @functools.partial(
        pl.kernel,
        mesh=mesh,
        out_type=jax.ShapeDtypeStruct((2, NP, LAT), f32),
        scratch_types=[
            pltpu.VMEM_SHARED((NP, LAT), f32),
            pltpu.VMEM((C,), jnp.int32),
            pltpu.VMEM((C, LAT), f32),
        ],
    )
    def sk(e, ridx, zrows, out, acc, iv, rv):
        cid = lax.axis_index("c")
        sid = lax.axis_index("s")
        w = sid * 2 + cid
        pltpu.sync_copy(zrows, acc.at[pl.ds(sid * RSUB, RSUB)])
        plsc.subcore_barrier()

        def body(j, carry):
            row = w * CH + j
            pltpu.sync_copy(ridx.at[row], iv)
            pltpu.sync_copy(e.at[pl.ds(row * C, C)], rv)
            pltpu.sync_copy(rv, acc.at[iv], add=True)
            return carry

        lax.fori_loop(0, CH, body, 0)
        plsc.subcore_barrier()
        pltpu.sync_copy(acc.at[pl.ds(sid * RSUB, RSUB)],
                        out.at[cid, pl.ds(sid * RSUB, RSUB)])

    return sk


def _gather_pairs128(ta, ia, tb, ib):
    return _make_gather2(LAT)(ta, ia, tb, ib)


def _scatter_partials(e, ridx, zrows):
    return _make_scatter()(e, ridx, zrows)


def _ln(y, g, be):
    mu = jnp.mean(y, axis=-1, keepdims=True)
    d = y - mu
    var = jnp.mean(d * d, axis=-1, keepdims=True)
    return d * lax.rsqrt(var + 1e-5) * g + be


def _full(shape):
    return pl.BlockSpec(shape, lambda i: (0, 0))


def _rows(rb, w):
    return pl.BlockSpec((rb, w), lambda i: (i, 0))


def _enc_body(x_ref, w0, b0, w1, b1, g, be, out_ref):
    h = jnp.maximum(
        jnp.dot(x_ref[...], w0[...], preferred_element_type=f32) + b0[...], 0.0)
    y = jnp.dot(h, w1[...], preferred_element_type=f32) + b1[...]
    out_ref[...] = _ln(y, g[...], be[...])


def _node_enc(x, w0, b0, w1, b1, g, be):
    rb = 1000
    return pl.pallas_call(
        _enc_body,
        grid=(N // rb,),
        in_specs=[_rows(rb, 16), _full((16, LAT)), _full((1, LAT)),
                  _full((LAT, LAT)), _full((1, LAT)), _full((1, LAT)),
                  _full((1, LAT))],
        out_specs=_rows(rb, LAT),
        out_shape=jax.ShapeDtypeStruct((N, LAT), f32),
    )(x, w0, b0, w1, b1, g, be)


def _edge_enc_body(gs_ref, gr_ref, w0, b0, w1, b1, g, be, out_ref):
    x = gs_ref[...] - gr_ref[...]
    lane = lax.broadcasted_iota(jnp.int32, x.shape, 1)
    x2 = x * x
    nm = jnp.sqrt(jnp.sum(jnp.where(lane < 3, x2, 0.0), axis=-1, keepdims=True))
    nw = jnp.sqrt(jnp.sum(jnp.where((lane >= 4) & (lane < 7), x2, 0.0),
                          axis=-1, keepdims=True))
    ef = jnp.where(lane == 3, nm, jnp.where(lane == 7, nw, x))
    h = jnp.maximum(
        jnp.dot(ef, w0[...], preferred_element_type=f32) + b0[...], 0.0)
    y = jnp.dot(h, w1[...], preferred_element_type=f32) + b1[...]
    out_ref[...] = _ln(y, g[...], be[...])


def _edge_enc(gs, gr, w0, b0, w1, b1, g, be):
    rb = 2048
    return pl.pallas_call(
        _edge_enc_body,
        grid=(EP // rb,),
        in_specs=[_rows(rb, LAT), _rows(rb, LAT), _full((LAT, LAT)),
                  _full((1, LAT)), _full((LAT, LAT)), _full((1, LAT)),
                  _full((1, LAT)), _full((1, LAT))],
        out_specs=_rows(rb, LAT),
        out_shape=jax.ShapeDtypeStruct((EP, LAT), f32),
    )(gs, gr, w0, b0, w1, b1, g, be)


def _proj_body(x_ref, ws, wr, p_ref, q_ref):
    x = x_ref[...]
    p_ref[...] = jnp.dot(x, ws[...], preferred_element_type=f32)
    q_ref[...] = jnp.dot(x, wr[...], preferred_element_type=f32)


def _proj(x, ws, wr):
    rb = 1000
    return pl.pallas_call(
        _proj_body,
        grid=(N // rb,),
        in_specs=[_rows(rb, LAT), _full((LAT, LAT)), _full((LAT, LAT))],
        out_specs=(_rows(rb, LAT), _rows(rb, LAT)),
        out_shape=(jax.ShapeDtypeStruct((N, LAT), f32),
                   jax.ShapeDtypeStruct((N, LAT), f32)),
    )(x, ws, wr)


def _edge_mlp_body(el_ref, gs_ref, gr_ref, w0, b0, w1, b1, g, be, out_ref):
    x = el_ref[...]
    h = jnp.maximum(
        jnp.dot(x, w0[...], preferred_element_type=f32)
        + gs_ref[...] + gr_ref[...] + b0[...], 0.0)
    y = jnp.dot(h, w1[...], preferred_element_type=f32) + b1[...]
    out_ref[...] = x + _ln(y, g[...], be[...])


def _edge_mlp(el, gs, gr, w0, b0, w1, b1, g, be):
    rb = 2048
    return pl.pallas_call(
        _edge_mlp_body,
        grid=(EP // rb,),
        in_specs=[_rows(rb, LAT), _rows(rb, LAT), _rows(rb, LAT),
                  _full((LAT, LAT)), _full((1, LAT)), _full((LAT, LAT)),
                  _full((1, LAT)), _full((1, LAT)), _full((1, LAT))],
        out_specs=_rows(rb, LAT),
        out_shape=jax.ShapeDtypeStruct((EP, LAT), f32),
    )(el, gs, gr, w0, b0, w1, b1, g, be)


def _node_mlp_body(nl_ref, a0_ref, a1_ref, wn, wa, b0, w1, b1, g, be, out_ref):
    x = nl_ref[...]
    a = a0_ref[...] + a1_ref[...]
    h = jnp.maximum(
        jnp.dot(x, wn[...], preferred_element_type=f32)
        + jnp.dot(a, wa[...], preferred_element_type=f32) + b0[...], 0.0)
    y = jnp.dot(h, w1[...], preferred_element_type=f32) + b1[...]
    out_ref[...] = x + _ln(y, g[...], be[...])


def _node_mlp(nl, a0, a1, wn, wa, b0, w1, b1, g, be):
    rb = 1000
    return pl.pallas_call(
        _node_mlp_body,
        grid=(N // rb,),
        in_specs=[_rows(rb, LAT), _rows(rb, LAT), _rows(rb, LAT),
                  _full((LAT, LAT)), _full((LAT, LAT)), _full((1, LAT)),
                  _full((LAT, LAT)), _full((1, LAT)), _full((1, LAT)),
                  _full((1, LAT))],
        out_specs=_rows(rb, LAT),
        out_shape=jax.ShapeDtypeStruct((N, LAT), f32),
    )(nl, a0, a1, wn, wa, b0, w1, b1, g, be)


def _dec_body(x_ref, w0, b0, w1, b1, out_ref):
    h = jnp.maximum(
        jnp.dot(x_ref[...], w0[...], preferred_element_type=f32) + b0[...], 0.0)
    out_ref[...] = jnp.dot(h, w1[...], preferred_element_type=f32) + b1[...]


def _decode(x, w0, b0, w1, b1):
    rb = 1000
    return pl.pallas_call(
        _dec_body,
        grid=(N // rb,),
        in_specs=[_rows(rb, LAT), _full((LAT, LAT)), _full((1, LAT)),
                  _full((LAT, LAT)), _full((1, LAT))],
        out_specs=_rows(rb, LAT),
        out_shape=jax.ShapeDtypeStruct((N, LAT), f32),
    )(x, w0, b0, w1, b1)


def kernel(world_pos, mesh_pos, target_world_pos, node_type, edge_index, params):
    senders = edge_index[0].astype(jnp.int32)
    receivers = edge_index[1].astype(jnp.int32)
    pad_e = EP - E
    zpad = jnp.zeros((pad_e,), jnp.int32)
    sp = jnp.concatenate([senders, zpad]).reshape(EP // C, C)
    rp = jnp.concatenate([receivers, zpad]).reshape(EP // C, C)
    dump = N + (jnp.arange(pad_e, dtype=jnp.int32) % (NP - N))
    r_scat = jnp.concatenate([receivers, dump]).reshape(EP // C, C)
    zrows = jnp.zeros((RSUB, LAT), f32)

    pos128 = jnp.zeros((N, LAT), f32)
    pos128 = pos128.at[:, 0:3].set(mesh_pos).at[:, 4:7].set(world_pos)

    nt = node_type[:, 0]
    one_hot = (nt[:, None] == jnp.arange(9)[None, :]).astype(f32)
    vel = jnp.where((nt == 1)[:, None], target_world_pos - world_pos, 0.0)
    nf16 = jnp.pad(jnp.concatenate([vel, one_hot], axis=-1), ((0, 0), (0, 4)))

    pn = params['enc_node']
    w0n = jnp.zeros((16, LAT), f32).at[0:12].set(pn['W0'])
    node_lat = _node_enc(nf16, w0n, pn['b0'].reshape(1, LAT),
                         pn['W1'], pn['b1'].reshape(1, LAT),
                         pn['g'].reshape(1, LAT), pn['be'].reshape(1, LAT))

    gps, gpr = _gather_pairs128(pos128, sp, pos128, rp)
    pe = params['enc_edge']
    w0e = jnp.zeros((LAT, LAT), f32).at[0:8].set(pe['W0'])
    edge_lat = _edge_enc(gps, gpr, w0e, pe['b0'].reshape(1, LAT),
                         pe['W1'], pe['b1'].reshape(1, LAT),
                         pe['g'].reshape(1, LAT), pe['be'].reshape(1, LAT))

    pew = params['proc_edge']
    pnw = params['proc_node']
    for i in range(STEPS):
        w0 = pew['W0'][i]
        p, q = _proj(node_lat, w0[LAT:2 * LAT], w0[2 * LAT:])
        gs, gr = _gather_pairs128(p, sp, q, rp)
        edge_lat = _edge_mlp(edge_lat, gs, gr, w0[:LAT],
                             pew['b0'][i].reshape(1, LAT), pew['W1'][i],
                             pew['b1'][i].reshape(1, LAT),
                             pew['g'][i].reshape(1, LAT),
                             pew['be'][i].reshape(1, LAT))
        parts = _scatter_partials(edge_lat, r_scat, zrows)
        nw0 = pnw['W0'][i]
        node_lat = _node_mlp(node_lat, parts[0, :N], parts[1, :N],
                             nw0[:LAT], nw0[LAT:],
                             pnw['b0'][i].reshape(1, LAT), pnw['W1'][i],
                             pnw['b1'][i].reshape(1, LAT),
                             pnw['g'][i].reshape(1, LAT),
                             pnw['be'][i].reshape(1, LAT))

    pd = params['dec']
    w1d = jnp.zeros((LAT, LAT), f32).at[:, 0:3].set(pd['W1'])
    b1d = jnp.zeros((1, LAT), f32).at[0, 0:3].set(pd['b1'])
    out = _decode(node_lat, pd['W0'], pd['b0'].reshape(1, LAT), w1d, b1d)
    return out[:, :3]


# speedup vs baseline: 2.4309x; 1.1333x over previous
import functools

import jax
import jax.numpy as jnp
from jax import lax
from jax.experimental import pallas as pl
from jax.experimental.pallas import tpu as pltpu
from jax.experimental.pallas import tpu_sc as plsc

N = 10000
E = 320000
LAT = 128
STEPS = 15
C = 128
NW = 32
EP = 327680
CH = EP // (NW * C)
NP = 10112
RSUB = NP // 16
f32 = jnp.float32


NBUF = 4
SNBUF = 2


@functools.lru_cache(maxsize=None)
def _make_gather2(d):
    mesh = plsc.VectorSubcoreMesh(core_axis_name="c", subcore_axis_name="s")
    G = (2 * CH) // NBUF

    @functools.partial(
        pl.kernel,
        mesh=mesh,
        out_type=(jax.ShapeDtypeStruct((EP, d), f32),
                  jax.ShapeDtypeStruct((EP, d), f32)),
        scratch_types=[
            pltpu.VMEM((2 * CH, C), jnp.int32),
            pltpu.VMEM((NBUF, C, d), f32),
            pltpu.SemaphoreType.DMA((NBUF,)),
            pltpu.SemaphoreType.DMA((NBUF,)),
        ],
    )
    def gk(ta, ia, tb, ib, oa, ob, idxall, rows, semg, semw):
        w = lax.axis_index("s") * 2 + lax.axis_index("c")
        pltpu.sync_copy(ia.at[pl.ds(w * CH, CH)], idxall.at[pl.ds(0, CH)])
        pltpu.sync_copy(ib.at[pl.ds(w * CH, CH)], idxall.at[pl.ds(CH, CH)])

        def body(g, carry):
            handles = []
            for b in range(NBUF):
                t = b % 2
                j = g * (NBUF // 2) + b // 2
                tab = ta if t == 0 else tb
                out = oa if t == 0 else ob
                base = (w * CH + j) * C

                @pl.when(g > 0)
                def _drain():
                    pltpu.make_async_copy(
                        rows.at[b], out.at[pl.ds(base, C)], semw.at[b]).wait()

                handles.append(pltpu.async_copy(
                    tab.at[idxall.at[t * CH + j]], rows.at[b], semg.at[b]))
            for b in range(NBUF):
                t = b % 2
                j = g * (NBUF // 2) + b // 2
                out = oa if t == 0 else ob
                base = (w * CH + j) * C
                handles[b].wait()
                pltpu.async_copy(rows.at[b], out.at[pl.ds(base, C)],
                                 semw.at[b])
            return carry

        lax.fori_loop(0, G, body, 0)
        for b in range(NBUF):
            pltpu.make_async_copy(
                rows.at[b], (oa if b % 2 == 0 else ob).at[pl.ds(0, C)],
                semw.at[b]).wait()

    return gk


@functools.lru_cache(maxsize=None)
def _make_scatter():
    mesh = plsc.VectorSubcoreMesh(core_axis_name="c", subcore_axis_name="s")

    @functools.partial(
        pl.kernel,
        mesh=mesh,
        out_type=jax.ShapeDtypeStruct((2, NP, LAT), f32),
        scratch_types=[
            pltpu.VMEM_SHARED((NP, LAT), f32),
            pltpu.VMEM((CH, C), jnp.int32),
            pltpu.VMEM((SNBUF, C, LAT), f32),
            pltpu.SemaphoreType.DMA((SNBUF,)),
        ],
    )
    def sk(e, ridx, zrows, out, acc, idxall, rows, seml):
        cid = lax.axis_index("c")
        sid = lax.axis_index("s")
        w = sid * 2 + cid
        pltpu.sync_copy(ridx.at[pl.ds(w * CH, CH)], idxall)
        pltpu.sync_copy(zrows, acc.at[pl.ds(sid * RSUB, RSUB)])
        plsc.subcore_barrier()

        G = CH // SNBUF
        for b in range(SNBUF):
            pltpu.async_copy(e.at[pl.ds((w * CH + b) * C, C)], rows.at[b],
                             seml.at[b])

        def body(g, carry):
            for b in range(SNBUF):
                j = g * SNBUF + b
                pltpu.make_async_copy(
                    e.at[pl.ds((w * CH + j) * C, C)], rows.at[b],
                    seml.at[b]).wait()
                pltpu.sync_copy(rows.at[b], acc.at[idxall.at[j]], add=True)

                @pl.when(g < G - 1)
                def _prefetch():
                    jn = j + SNBUF
                    pltpu.async_copy(e.at[pl.ds((w * CH + jn) * C, C)],
                                     rows.at[b], seml.at[b])
            return carry

        lax.fori_loop(0, G, body, 0)
        plsc.subcore_barrier()
        pltpu.sync_copy(acc.at[pl.ds(sid * RSUB, RSUB)],
                        out.at[cid, pl.ds(sid * RSUB, RSUB)])

    return sk


def _gather_pairs128(ta, ia, tb, ib):
    return _make_gather2(LAT)(ta, ia, tb, ib)


def _scatter_partials(e, ridx, zrows):
    return _make_scatter()(e, ridx, zrows)


def _ln(y, g, be):
    mu = jnp.mean(y, axis=-1, keepdims=True)
    d = y - mu
    var = jnp.mean(d * d, axis=-1, keepdims=True)
    return d * lax.rsqrt(var + 1e-5) * g + be


def _full(shape):
    return pl.BlockSpec(shape, lambda i: (0, 0))


def _rows(rb, w):
    return pl.BlockSpec((rb, w), lambda i: (i, 0))


def _enc_body(x_ref, w0, b0, w1, b1, g, be, out_ref):
    h = jnp.maximum(
        jnp.dot(x_ref[...], w0[...], preferred_element_type=f32) + b0[...], 0.0)
    y = jnp.dot(h, w1[...], preferred_element_type=f32) + b1[...]
    out_ref[...] = _ln(y, g[...], be[...])


def _node_enc(x, w0, b0, w1, b1, g, be):
    rb = 1000
    return pl.pallas_call(
        _enc_body,
        grid=(N // rb,),
        in_specs=[_rows(rb, 16), _full((16, LAT)), _full((1, LAT)),
                  _full((LAT, LAT)), _full((1, LAT)), _full((1, LAT)),
                  _full((1, LAT))],
        out_specs=_rows(rb, LAT),
        out_shape=jax.ShapeDtypeStruct((N, LAT), f32),
    )(x, w0, b0, w1, b1, g, be)


def _edge_enc_body(gs_ref, gr_ref, w0, b0, w1, b1, g, be, out_ref):
    x = gs_ref[...] - gr_ref[...]
    lane = lax.broadcasted_iota(jnp.int32, x.shape, 1)
    x2 = x * x
    nm = jnp.sqrt(jnp.sum(jnp.where(lane < 3, x2, 0.0), axis=-1, keepdims=True))
    nw = jnp.sqrt(jnp.sum(jnp.where((lane >= 4) & (lane < 7), x2, 0.0),
                          axis=-1, keepdims=True))
    ef = jnp.where(lane == 3, nm, jnp.where(lane == 7, nw, x))
    h = jnp.maximum(
        jnp.dot(ef, w0[...], preferred_element_type=f32) + b0[...], 0.0)
    y = jnp.dot(h, w1[...], preferred_element_type=f32) + b1[...]
    out_ref[...] = _ln(y, g[...], be[...])


def _edge_enc(gs, gr, w0, b0, w1, b1, g, be):
    rb = 2048
    return pl.pallas_call(
        _edge_enc_body,
        grid=(EP // rb,),
        in_specs=[_rows(rb, LAT), _rows(rb, LAT), _full((LAT, LAT)),
                  _full((1, LAT)), _full((LAT, LAT)), _full((1, LAT)),
                  _full((1, LAT)), _full((1, LAT))],
        out_specs=_rows(rb, LAT),
        out_shape=jax.ShapeDtypeStruct((EP, LAT), f32),
    )(gs, gr, w0, b0, w1, b1, g, be)


def _proj_body(x_ref, ws, wr, p_ref, q_ref):
    x = x_ref[...]
    p_ref[...] = jnp.dot(x, ws[...], preferred_element_type=f32)
    q_ref[...] = jnp.dot(x, wr[...], preferred_element_type=f32)


def _proj(x, ws, wr):
    rb = 1000
    return pl.pallas_call(
        _proj_body,
        grid=(N // rb,),
        in_specs=[_rows(rb, LAT), _full((LAT, LAT)), _full((LAT, LAT))],
        out_specs=(_rows(rb, LAT), _rows(rb, LAT)),
        out_shape=(jax.ShapeDtypeStruct((N, LAT), f32),
                   jax.ShapeDtypeStruct((N, LAT), f32)),
    )(x, ws, wr)


def _edge_mlp_body(el_ref, gs_ref, gr_ref, w0, b0, w1, b1, g, be, out_ref):
    x = el_ref[...]
    h = jnp.maximum(
        jnp.dot(x, w0[...], preferred_element_type=f32)
        + gs_ref[...] + gr_ref[...] + b0[...], 0.0)
    y = jnp.dot(h, w1[...], preferred_element_type=f32) + b1[...]
    out_ref[...] = x + _ln(y, g[...], be[...])


def _edge_mlp(el, gs, gr, w0, b0, w1, b1, g, be):
    rb = 2048
    return pl.pallas_call(
        _edge_mlp_body,
        grid=(EP // rb,),
        in_specs=[_rows(rb, LAT), _rows(rb, LAT), _rows(rb, LAT),
                  _full((LAT, LAT)), _full((1, LAT)), _full((LAT, LAT)),
                  _full((1, LAT)), _full((1, LAT)), _full((1, LAT))],
        out_specs=_rows(rb, LAT),
        out_shape=jax.ShapeDtypeStruct((EP, LAT), f32),
    )(el, gs, gr, w0, b0, w1, b1, g, be)


def _node_mlp_body(nl_ref, a0_ref, a1_ref, wn, wa, b0, w1, b1, g, be, out_ref):
    x = nl_ref[...]
    a = a0_ref[...] + a1_ref[...]
    h = jnp.maximum(
        jnp.dot(x, wn[...], preferred_element_type=f32)
        + jnp.dot(a, wa[...], preferred_element_type=f32) + b0[...], 0.0)
    y = jnp.dot(h, w1[...], preferred_element_type=f32) + b1[...]
    out_ref[...] = x + _ln(y, g[...], be[...])


def _node_mlp(nl, a0, a1, wn, wa, b0, w1, b1, g, be):
    rb = 1000
    return pl.pallas_call(
        _node_mlp_body,
        grid=(N // rb,),
        in_specs=[_rows(rb, LAT), _rows(rb, LAT), _rows(rb, LAT),
                  _full((LAT, LAT)), _full((LAT, LAT)), _full((1, LAT)),
                  _full((LAT, LAT)), _full((1, LAT)), _full((1, LAT)),
                  _full((1, LAT))],
        out_specs=_rows(rb, LAT),
        out_shape=jax.ShapeDtypeStruct((N, LAT), f32),
    )(nl, a0, a1, wn, wa, b0, w1, b1, g, be)


def _dec_body(x_ref, w0, b0, w1, b1, out_ref):
    h = jnp.maximum(
        jnp.dot(x_ref[...], w0[...], preferred_element_type=f32) + b0[...], 0.0)
    out_ref[...] = jnp.dot(h, w1[...], preferred_element_type=f32) + b1[...]


def _decode(x, w0, b0, w1, b1):
    rb = 1000
    return pl.pallas_call(
        _dec_body,
        grid=(N // rb,),
        in_specs=[_rows(rb, LAT), _full((LAT, LAT)), _full((1, LAT)),
                  _full((LAT, LAT)), _full((1, LAT))],
        out_specs=_rows(rb, LAT),
        out_shape=jax.ShapeDtypeStruct((N, LAT), f32),
    )(x, w0, b0, w1, b1)


def kernel(world_pos, mesh_pos, target_world_pos, node_type, edge_index, params):
    senders = edge_index[0].astype(jnp.int32)
    receivers = edge_index[1].astype(jnp.int32)
    pad_e = EP - E
    zpad = jnp.zeros((pad_e,), jnp.int32)
    sp = jnp.concatenate([senders, zpad]).reshape(EP // C, C)
    rp = jnp.concatenate([receivers, zpad]).reshape(EP // C, C)
    dump = N + (jnp.arange(pad_e, dtype=jnp.int32) % (NP - N))
    r_scat = jnp.concatenate([receivers, dump]).reshape(EP // C, C)
    zrows = jnp.zeros((RSUB, LAT), f32)

    pos128 = jnp.zeros((N, LAT), f32)
    pos128 = pos128.at[:, 0:3].set(mesh_pos).at[:, 4:7].set(world_pos)

    nt = node_type[:, 0]
    one_hot = (nt[:, None] == jnp.arange(9)[None, :]).astype(f32)
    vel = jnp.where((nt == 1)[:, None], target_world_pos - world_pos, 0.0)
    nf16 = jnp.pad(jnp.concatenate([vel, one_hot], axis=-1), ((0, 0), (0, 4)))

    pn = params['enc_node']
    w0n = jnp.zeros((16, LAT), f32).at[0:12].set(pn['W0'])
    node_lat = _node_enc(nf16, w0n, pn['b0'].reshape(1, LAT),
                         pn['W1'], pn['b1'].reshape(1, LAT),
                         pn['g'].reshape(1, LAT), pn['be'].reshape(1, LAT))

    gps, gpr = _gather_pairs128(pos128, sp, pos128, rp)
    pe = params['enc_edge']
    w0e = jnp.zeros((LAT, LAT), f32).at[0:8].set(pe['W0'])
    edge_lat = _edge_enc(gps, gpr, w0e, pe['b0'].reshape(1, LAT),
                         pe['W1'], pe['b1'].reshape(1, LAT),
                         pe['g'].reshape(1, LAT), pe['be'].reshape(1, LAT))

    pew = params['proc_edge']
    pnw = params['proc_node']
    for i in range(STEPS):
        w0 = pew['W0'][i]
        p, q = _proj(node_lat, w0[LAT:2 * LAT], w0[2 * LAT:])
        gs, gr = _gather_pairs128(p, sp, q, rp)
        edge_lat = _edge_mlp(edge_lat, gs, gr, w0[:LAT],
                             pew['b0'][i].reshape(1, LAT), pew['W1'][i],
                             pew['b1'][i].reshape(1, LAT),
                             pew['g'][i].reshape(1, LAT),
                             pew['be'][i].reshape(1, LAT))
        parts = _scatter_partials(edge_lat, r_scat, zrows)
        nw0 = pnw['W0'][i]
        node_lat = _node_mlp(node_lat, parts[0, :N], parts[1, :N],
                             nw0[:LAT], nw0[LAT:],
                             pnw['b0'][i].reshape(1, LAT), pnw['W1'][i],
                             pnw['b1'][i].reshape(1, LAT),
                             pnw['g'][i].reshape(1, LAT),
                             pnw['be'][i].reshape(1, LAT))

    pd = params['dec']
    w1d = jnp.zeros((LAT, LAT), f32).at[:, 0:3].set(pd['W1'])
    b1d = jnp.zeros((1, LAT), f32).at[0, 0:3].set(pd['b1'])
    out = _decode(node_lat, pd['W0'], pd['b0'].reshape(1, LAT), w1d, b1d)
    return out[:, :3]


# speedup vs baseline: 4.7826x; 1.9674x over previous
import functools

import jax
import jax.numpy as jnp
from jax import lax
from jax.experimental import pallas as pl
from jax.experimental.pallas import tpu as pltpu
from jax.experimental.pallas import tpu_sc as plsc

N = 10000
E = 320000
LAT = 128
STEPS = 15
C = 128
NW = 32
EP = 327680
CH = EP // (NW * C)
NP = 10112
RSUB = NP // 16
f32 = jnp.float32


NBUF = 4
SNBUF = 2


@functools.lru_cache(maxsize=None)
def _make_gather2(d, ne):
    mesh = plsc.VectorSubcoreMesh(core_axis_name="c", subcore_axis_name="s")
    chw = ne // (NW * C)
    G = (2 * chw) // NBUF

    @functools.partial(
        pl.kernel,
        mesh=mesh,
        out_type=(jax.ShapeDtypeStruct((ne, d), f32),
                  jax.ShapeDtypeStruct((ne, d), f32)),
        scratch_types=[
            pltpu.VMEM((2 * chw, C), jnp.int32),
            pltpu.VMEM((NBUF, C, d), f32),
            pltpu.SemaphoreType.DMA((NBUF,)),
            pltpu.SemaphoreType.DMA((NBUF,)),
        ],
    )
    def gk(ta, ia, tb, ib, oa, ob, idxall, rows, semg, semw):
        w = lax.axis_index("s") * 2 + lax.axis_index("c")
        pltpu.sync_copy(ia.at[pl.ds(w * chw, chw)], idxall.at[pl.ds(0, chw)])
        pltpu.sync_copy(ib.at[pl.ds(w * chw, chw)],
                        idxall.at[pl.ds(chw, chw)])

        def body(g, carry):
            handles = []
            for b in range(NBUF):
                t = b % 2
                j = g * (NBUF // 2) + b // 2
                tab = ta if t == 0 else tb
                out = oa if t == 0 else ob
                base = (w * chw + j) * C

                @pl.when(g > 0)
                def _drain():
                    pltpu.make_async_copy(
                        rows.at[b], out.at[pl.ds(base, C)], semw.at[b]).wait()

                handles.append(pltpu.async_copy(
                    tab.at[idxall.at[t * chw + j]], rows.at[b], semg.at[b]))
            for b in range(NBUF):
                t = b % 2
                j = g * (NBUF // 2) + b // 2
                out = oa if t == 0 else ob
                base = (w * chw + j) * C
                handles[b].wait()
                pltpu.async_copy(rows.at[b], out.at[pl.ds(base, C)],
                                 semw.at[b])
            return carry

        lax.fori_loop(0, G, body, 0)
        for b in range(NBUF):
            pltpu.make_async_copy(
                rows.at[b], (oa if b % 2 == 0 else ob).at[pl.ds(0, C)],
                semw.at[b]).wait()

    return gk


@functools.lru_cache(maxsize=None)
def _make_scatter(ne):
    mesh = plsc.VectorSubcoreMesh(core_axis_name="c", subcore_axis_name="s")
    chw = ne // (NW * C)

    @functools.partial(
        pl.kernel,
        mesh=mesh,
        out_type=jax.ShapeDtypeStruct((2, NP, LAT), f32),
        scratch_types=[
            pltpu.VMEM_SHARED((NP, LAT), f32),
            pltpu.VMEM((2 * chw, C), jnp.int32),
            pltpu.VMEM((SNBUF, C, LAT), f32),
            pltpu.SemaphoreType.DMA((SNBUF,)),
        ],
    )
    def sk(e1, ridx1, e2, ridx2, zrows, out, acc, idxall, rows, seml):
        cid = lax.axis_index("c")
        sid = lax.axis_index("s")
        w = sid * 2 + cid
        pltpu.sync_copy(ridx1.at[pl.ds(w * chw, chw)],
                        idxall.at[pl.ds(0, chw)])
        pltpu.sync_copy(ridx2.at[pl.ds(w * chw, chw)],
                        idxall.at[pl.ds(chw, chw)])
        pltpu.sync_copy(zrows, acc.at[pl.ds(sid * RSUB, RSUB)])
        plsc.subcore_barrier()

        G = chw // SNBUF
        for h, e in ((0, e1), (1, e2)):
            for b in range(SNBUF):
                pltpu.async_copy(e.at[pl.ds((w * chw + b) * C, C)],
                                 rows.at[b], seml.at[b])

            def body(g, carry, e=e, h=h):
                for b in range(SNBUF):
                    j = g * SNBUF + b
                    pltpu.make_async_copy(
                        e.at[pl.ds((w * chw + j) * C, C)], rows.at[b],
                        seml.at[b]).wait()
                    pltpu.sync_copy(rows.at[b],
                                    acc.at[idxall.at[h * chw + j]], add=True)

                    @pl.when(g < G - 1)
                    def _prefetch():
                        jn = j + SNBUF
                        pltpu.async_copy(e.at[pl.ds((w * chw + jn) * C, C)],
                                         rows.at[b], seml.at[b])
                return carry

            lax.fori_loop(0, G, body, 0)
        plsc.subcore_barrier()
        pltpu.sync_copy(acc.at[pl.ds(sid * RSUB, RSUB)],
                        out.at[cid, pl.ds(sid * RSUB, RSUB)])

    return sk


def _gather_pairs128(ta, ia, tb, ib):
    return _make_gather2(LAT, ia.shape[0] * C)(ta, ia, tb, ib)


def _scatter_partials(e1, r1, e2, r2, zrows):
    return _make_scatter(e1.shape[0])(e1, r1, e2, r2, zrows)


def _ln(y, g, be):
    mu = jnp.mean(y, axis=-1, keepdims=True)
    d = y - mu
    var = jnp.mean(d * d, axis=-1, keepdims=True)
    return d * lax.rsqrt(var + 1e-5) * g + be


def _full(shape):
    return pl.BlockSpec(shape, lambda i: (0, 0))


def _rows(rb, w):
    return pl.BlockSpec((rb, w), lambda i: (i, 0))


def _enc_body(x_ref, w0, b0, w1, b1, g, be, out_ref):
    h = jnp.maximum(
        jnp.dot(x_ref[...], w0[...], preferred_element_type=f32) + b0[...], 0.0)
    y = jnp.dot(h, w1[...], preferred_element_type=f32) + b1[...]
    out_ref[...] = _ln(y, g[...], be[...])


def _node_enc(x, w0, b0, w1, b1, g, be):
    rb = 1000
    return pl.pallas_call(
        _enc_body,
        grid=(N // rb,),
        in_specs=[_rows(rb, 16), _full((16, LAT)), _full((1, LAT)),
                  _full((LAT, LAT)), _full((1, LAT)), _full((1, LAT)),
                  _full((1, LAT))],
        out_specs=_rows(rb, LAT),
        out_shape=jax.ShapeDtypeStruct((N, LAT), f32),
    )(x, w0, b0, w1, b1, g, be)


def _edge_enc_body(gs_ref, gr_ref, w0, b0, w1, b1, g, be, out_ref):
    x = gs_ref[...] - gr_ref[...]
    lane = lax.broadcasted_iota(jnp.int32, x.shape, 1)
    x2 = x * x
    nm = jnp.sqrt(jnp.sum(jnp.where(lane < 3, x2, 0.0), axis=-1, keepdims=True))
    nw = jnp.sqrt(jnp.sum(jnp.where((lane >= 4) & (lane < 7), x2, 0.0),
                          axis=-1, keepdims=True))
    ef = jnp.where(lane == 3, nm, jnp.where(lane == 7, nw, x))
    h = jnp.maximum(
        jnp.dot(ef, w0[...], preferred_element_type=f32) + b0[...], 0.0)
    y = jnp.dot(h, w1[...], preferred_element_type=f32) + b1[...]
    out_ref[...] = _ln(y, g[...], be[...])


def _edge_enc(gs, gr, w0, b0, w1, b1, g, be):
    rb = 2048
    return pl.pallas_call(
        _edge_enc_body,
        grid=(gs.shape[0] // rb,),
        in_specs=[_rows(rb, LAT), _rows(rb, LAT), _full((LAT, LAT)),
                  _full((1, LAT)), _full((LAT, LAT)), _full((1, LAT)),
                  _full((1, LAT)), _full((1, LAT))],
        out_specs=_rows(rb, LAT),
        out_shape=jax.ShapeDtypeStruct((gs.shape[0], LAT), f32),
    )(gs, gr, w0, b0, w1, b1, g, be)


def _proj_body(x_ref, ws, wr, p_ref, q_ref):
    x = x_ref[...]
    p_ref[...] = jnp.dot(x, ws[...], preferred_element_type=f32)
    q_ref[...] = jnp.dot(x, wr[...], preferred_element_type=f32)


def _proj(x, ws, wr):
    rb = 1000
    return pl.pallas_call(
        _proj_body,
        grid=(N // rb,),
        in_specs=[_rows(rb, LAT), _full((LAT, LAT)), _full((LAT, LAT))],
        out_specs=(_rows(rb, LAT), _rows(rb, LAT)),
        out_shape=(jax.ShapeDtypeStruct((N, LAT), f32),
                   jax.ShapeDtypeStruct((N, LAT), f32)),
    )(x, ws, wr)


def _edge_mlp_body(el_ref, gs_ref, gr_ref, w0, b0, w1, b1, g, be, out_ref):
    x = el_ref[...]
    h = jnp.maximum(
        jnp.dot(x, w0[...], preferred_element_type=f32)
        + gs_ref[...] + gr_ref[...] + b0[...], 0.0)
    y = jnp.dot(h, w1[...], preferred_element_type=f32) + b1[...]
    out_ref[...] = x + _ln(y, g[...], be[...])


def _edge_mlp(el, gs, gr, w0, b0, w1, b1, g, be):
    rb = 2048
    return pl.pallas_call(
        _edge_mlp_body,
        grid=(el.shape[0] // rb,),
        in_specs=[_rows(rb, LAT), _rows(rb, LAT), _rows(rb, LAT),
                  _full((LAT, LAT)), _full((1, LAT)), _full((LAT, LAT)),
                  _full((1, LAT)), _full((1, LAT)), _full((1, LAT))],
        out_specs=_rows(rb, LAT),
        out_shape=jax.ShapeDtypeStruct((el.shape[0], LAT), f32),
    )(el, gs, gr, w0, b0, w1, b1, g, be)


def _node_mlp_body(nl_ref, a0_ref, a1_ref, wn, wa, b0, w1, b1, g, be, out_ref):
    x = nl_ref[...]
    a = a0_ref[...] + a1_ref[...]
    h = jnp.maximum(
        jnp.dot(x, wn[...], preferred_element_type=f32)
        + jnp.dot(a, wa[...], preferred_element_type=f32) + b0[...], 0.0)
    y = jnp.dot(h, w1[...], preferred_element_type=f32) + b1[...]
    out_ref[...] = x + _ln(y, g[...], be[...])


def _node_mlp(nl, a0, a1, wn, wa, b0, w1, b1, g, be):
    rb = 1000
    return pl.pallas_call(
        _node_mlp_body,
        grid=(N // rb,),
        in_specs=[_rows(rb, LAT), _rows(rb, LAT), _rows(rb, LAT),
                  _full((LAT, LAT)), _full((LAT, LAT)), _full((1, LAT)),
                  _full((LAT, LAT)), _full((1, LAT)), _full((1, LAT)),
                  _full((1, LAT))],
        out_specs=_rows(rb, LAT),
        out_shape=jax.ShapeDtypeStruct((N, LAT), f32),
    )(nl, a0, a1, wn, wa, b0, w1, b1, g, be)


def _dec_body(x_ref, w0, b0, w1, b1, out_ref):
    h = jnp.maximum(
        jnp.dot(x_ref[...], w0[...], preferred_element_type=f32) + b0[...], 0.0)
    out_ref[...] = jnp.dot(h, w1[...], preferred_element_type=f32) + b1[...]


def _decode(x, w0, b0, w1, b1):
    rb = 1000
    return pl.pallas_call(
        _dec_body,
        grid=(N // rb,),
        in_specs=[_rows(rb, LAT), _full((LAT, LAT)), _full((1, LAT)),
                  _full((LAT, LAT)), _full((1, LAT))],
        out_specs=_rows(rb, LAT),
        out_shape=jax.ShapeDtypeStruct((N, LAT), f32),
    )(x, w0, b0, w1, b1)


def kernel(world_pos, mesh_pos, target_world_pos, node_type, edge_index, params):
    senders = edge_index[0].astype(jnp.int32)
    receivers = edge_index[1].astype(jnp.int32)
    pad_e = EP - E
    zpad = jnp.arange(pad_e, dtype=jnp.int32) % N
    sp = jnp.concatenate([senders, zpad]).reshape(EP // C, C)
    rp = jnp.concatenate([receivers, zpad]).reshape(EP // C, C)
    dump = N + (jnp.arange(pad_e, dtype=jnp.int32) % (NP - N))
    r_scat = jnp.concatenate([receivers, dump]).reshape(EP // C, C)
    zrows = jnp.zeros((RSUB, LAT), f32)
    RH = (EP // 2) // C
    sp1, sp2 = sp[:RH], sp[RH:]
    rp1, rp2 = rp[:RH], rp[RH:]
    rs1, rs2 = r_scat[:RH], r_scat[RH:]

    pos128 = jnp.zeros((N, LAT), f32)
    pos128 = pos128.at[:, 0:3].set(mesh_pos).at[:, 4:7].set(world_pos)

    nt = node_type[:, 0]
    one_hot = (nt[:, None] == jnp.arange(9)[None, :]).astype(f32)
    vel = jnp.where((nt == 1)[:, None], target_world_pos - world_pos, 0.0)
    nf16 = jnp.pad(jnp.concatenate([vel, one_hot], axis=-1), ((0, 0), (0, 4)))

    pn = params['enc_node']
    w0n = jnp.zeros((16, LAT), f32).at[0:12].set(pn['W0'])
    node_lat = _node_enc(nf16, w0n, pn['b0'].reshape(1, LAT),
                         pn['W1'], pn['b1'].reshape(1, LAT),
                         pn['g'].reshape(1, LAT), pn['be'].reshape(1, LAT))

    pe = params['enc_edge']
    w0e = jnp.zeros((LAT, LAT), f32).at[0:8].set(pe['W0'])
    enc_args = (w0e, pe['b0'].reshape(1, LAT), pe['W1'],
                pe['b1'].reshape(1, LAT), pe['g'].reshape(1, LAT),
                pe['be'].reshape(1, LAT))
    g1s, g1r = _gather_pairs128(pos128, sp1, pos128, rp1)
    el1 = _edge_enc(g1s, g1r, *enc_args)
    g2s, g2r = _gather_pairs128(pos128, sp2, pos128, rp2)
    el2 = _edge_enc(g2s, g2r, *enc_args)

    pew = params['proc_edge']
    pnw = params['proc_node']
    for i in range(STEPS):
        w0 = pew['W0'][i]
        p, q = _proj(node_lat, w0[LAT:2 * LAT], w0[2 * LAT:])
        mlp_args = (w0[:LAT], pew['b0'][i].reshape(1, LAT), pew['W1'][i],
                    pew['b1'][i].reshape(1, LAT),
                    pew['g'][i].reshape(1, LAT),
                    pew['be'][i].reshape(1, LAT))
        gs1, gr1 = _gather_pairs128(p, sp1, q, rp1)
        el1 = _edge_mlp(el1, gs1, gr1, *mlp_args)
        gs2, gr2 = _gather_pairs128(p, sp2, q, rp2)
        el2 = _edge_mlp(el2, gs2, gr2, *mlp_args)
        parts = _scatter_partials(el1, rs1, el2, rs2, zrows)
        nw0 = pnw['W0'][i]
        node_lat = _node_mlp(node_lat, parts[0, :N], parts[1, :N],
                             nw0[:LAT], nw0[LAT:],
                             pnw['b0'][i].reshape(1, LAT), pnw['W1'][i],
                             pnw['b1'][i].reshape(1, LAT),
                             pnw['g'][i].reshape(1, LAT),
                             pnw['be'][i].reshape(1, LAT))

    pd = params['dec']
    w1d = jnp.zeros((LAT, LAT), f32).at[:, 0:3].set(pd['W1'])
    b1d = jnp.zeros((1, LAT), f32).at[0, 0:3].set(pd['b1'])
    out = _decode(node_lat, pd['W0'], pd['b0'].reshape(1, LAT), w1d, b1d)
    return out[:, :3]
